# baseline (speedup 1.0000x reference)
import functools
import math

import jax
import jax.numpy as jnp
from jax import lax
from jax.experimental import pallas as pl
from jax.experimental.pallas import tpu as pltpu
from jax.experimental.pallas import tpu_sc as plsc

D_MODEL_K = 1024
SCALE_K = math.sqrt(D_MODEL_K)

_INFO = plsc.get_sparse_core_info()
_NC = _INFO.num_cores
_NS = _INFO.num_subcores
_NW = _NC * _NS
_LANES = _INFO.num_lanes

_CHUNK = 8
_NBUF = 10


def _scale_buf(rows_ref):
  def row_body(r, _):
    for j in range(D_MODEL_K // _LANES):
      rows_ref[r, pl.ds(j * _LANES, _LANES)] = (
          rows_ref[r, pl.ds(j * _LANES, _LANES)] * jnp.float32(SCALE_K)
      )
    return 0
  lax.fori_loop(0, _CHUNK, row_body, 0)


def _emb_kernel(table_hbm, idx_hbm, out_hbm, idx_v, *bufs_and_sems,
                b_per_w, n_chunks):
  bufs = bufs_and_sems[:_NBUF]
  gsems = bufs_and_sems[_NBUF:2 * _NBUF]
  ssems = bufs_and_sems[2 * _NBUF:3 * _NBUF]

  wid = lax.axis_index("s") * _NC + lax.axis_index("c")
  base = wid * b_per_w
  pltpu.sync_copy(idx_hbm.at[pl.ds(base, b_per_w)], idx_v)

  def gather_start(c, b):
    pltpu.async_copy(
        table_hbm.at[idx_v.at[pl.ds(c * _CHUNK, _CHUNK)]], bufs[b], gsems[b])

  def store_start(c, b):
    pltpu.async_copy(
        bufs[b], out_hbm.at[pl.ds(base + c * _CHUNK, _CHUNK)], ssems[b])

  def wait_gather(b):
    pltpu.make_async_copy(
        table_hbm.at[pl.ds(0, _CHUNK)], bufs[b], gsems[b]).wait()

  def wait_store(b):
    pltpu.make_async_copy(
        bufs[b], out_hbm.at[pl.ds(base, _CHUNK)], ssems[b]).wait()

  lead = _NBUF - 2

  def when(cond, fn):
    if isinstance(cond, bool):
      if cond:
        fn()
    else:
      pl.when(cond)(fn)

  for c in range(min(lead, n_chunks)):
    gather_start(c, c % _NBUF)

  def step(i, b):
    refill = i + lead
    rb = (b + lead) % _NBUF

    def do_refill():
      when(refill >= _NBUF,
           lambda: wait_store(rb))
      gather_start(refill, rb)

    when(refill < n_chunks, do_refill)

    wait_gather(b)
    _scale_buf(bufs[b])
    store_start(i, b)

  def ring_body(g, _):
    for b in range(_NBUF):
      step(_NBUF * g + b, b)
    return 0

  n_full = n_chunks // _NBUF
  lax.fori_loop(0, n_full, ring_body, 0)
  for i in range(n_full * _NBUF, n_chunks):
    step(i, i % _NBUF)
  for b in range(_NBUF):
    wait_store(b)


def kernel(x, table):
  batch, seq = x.shape
  n_tokens = batch * seq
  d_model = table.shape[1]
  assert n_tokens % (_NW * _CHUNK) == 0
  b_per_w = n_tokens // _NW
  n_chunks = b_per_w // _CHUNK

  idx = x.reshape(n_tokens).astype(jnp.int32)

  mesh = plsc.VectorSubcoreMesh(core_axis_name="c", subcore_axis_name="s")
  run = pl.kernel(
      functools.partial(_emb_kernel, b_per_w=b_per_w, n_chunks=n_chunks),
      mesh=mesh,
      out_type=jax.ShapeDtypeStruct((n_tokens, d_model), jnp.float32),
      scratch_types=(
          [pltpu.VMEM((b_per_w,), jnp.int32)]
          + [pltpu.VMEM((_CHUNK, d_model), jnp.float32)] * _NBUF
          + [pltpu.SemaphoreType.DMA] * (2 * _NBUF)
      ),
  )
  out = run(table, idx)
  return out.reshape(batch, seq, d_model)


# speedup vs baseline: 1.0496x; 1.0496x over previous
import functools
import math

import jax
import jax.numpy as jnp
from jax import lax
from jax.experimental import pallas as pl
from jax.experimental.pallas import tpu as pltpu
from jax.experimental.pallas import tpu_sc as plsc

D_MODEL_K = 1024
SCALE_K = math.sqrt(D_MODEL_K)

_INFO = plsc.get_sparse_core_info()
_NC = _INFO.num_cores
_NS = _INFO.num_subcores
_NW = _NC * _NS
_LANES = _INFO.num_lanes

_CHUNK = 8
_NBUF = 8


def _scale_buf(rows_ref):
  def row_body(r, _):
    for j in range(D_MODEL_K // _LANES):
      rows_ref[r, pl.ds(j * _LANES, _LANES)] = (
          rows_ref[r, pl.ds(j * _LANES, _LANES)] * jnp.float32(SCALE_K)
      )
    return 0
  lax.fori_loop(0, _CHUNK, row_body, 0)


def _emb_kernel(table_hbm, idx_hbm, out_hbm, idx_v, bufs, gsems, ssems,
                *, b_per_w, n_chunks):
  wid = lax.axis_index("s") * _NC + lax.axis_index("c")
  base = wid * b_per_w
  pltpu.sync_copy(idx_hbm.at[pl.ds(base, b_per_w)], idx_v)

  def gather_start(c, b):
    pltpu.async_copy(
        table_hbm.at[idx_v.at[pl.ds(c * _CHUNK, _CHUNK)]], bufs.at[b],
        gsems.at[b])

  def store_start(c, b):
    pltpu.async_copy(
        bufs.at[b], out_hbm.at[pl.ds(base + c * _CHUNK, _CHUNK)], ssems.at[b])

  def wait_gather(b):
    pltpu.make_async_copy(
        table_hbm.at[pl.ds(0, _CHUNK)], bufs.at[b], gsems.at[b]).wait()

  def wait_store(b):
    pltpu.make_async_copy(
        bufs.at[b], out_hbm.at[pl.ds(base, _CHUNK)], ssems.at[b]).wait()

  lead = _NBUF - 2

  def when(cond, fn):
    if isinstance(cond, bool):
      if cond:
        fn()
    else:
      pl.when(cond)(fn)

  for c in range(min(lead, n_chunks)):
    gather_start(c, c % _NBUF)

  def step(i, b):
    refill = i + lead
    rb = (b + lead) % _NBUF

    def do_refill():
      when(refill >= _NBUF,
           lambda: wait_store(rb))
      gather_start(refill, rb)

    when(refill < n_chunks, do_refill)

    wait_gather(b)
    _scale_buf(bufs.at[b])
    store_start(i, b)

  def ring_body(g, _):
    for b in range(_NBUF):
      step(_NBUF * g + b, b)
    return 0

  n_full = n_chunks // _NBUF
  lax.fori_loop(0, n_full, ring_body, 0)
  for i in range(n_full * _NBUF, n_chunks):
    step(i, i % _NBUF)
  for b in range(_NBUF):
    wait_store(b)


def kernel(x, table):
  batch, seq = x.shape
  n_tokens = batch * seq
  d_model = table.shape[1]
  assert n_tokens % (_NW * _CHUNK) == 0
  b_per_w = n_tokens // _NW
  n_chunks = b_per_w // _CHUNK

  idx = x.reshape(n_tokens).astype(jnp.int32)

  mesh = plsc.VectorSubcoreMesh(core_axis_name="c", subcore_axis_name="s")
  run = pl.kernel(
      functools.partial(_emb_kernel, b_per_w=b_per_w, n_chunks=n_chunks),
      mesh=mesh,
      out_type=jax.ShapeDtypeStruct((n_tokens, d_model), jnp.float32),
      scratch_types=[
          pltpu.VMEM((b_per_w,), jnp.int32),
          pltpu.VMEM((_NBUF, _CHUNK, d_model), jnp.float32),
          pltpu.SemaphoreType.DMA((_NBUF,)),
          pltpu.SemaphoreType.DMA((_NBUF,)),
      ],
  )
  out = run(table, idx)
  return out.reshape(batch, seq, d_model)
